# baseline (speedup 1.0000x reference)
import jax
import jax.numpy as jnp
from jax import lax
from jax.experimental import pallas as pl
from jax.experimental.pallas import tpu as pltpu
from jax.experimental.pallas import tpu_sc as plsc

BATCH = 4096
SEQ = 200
EMBED = 64
LANES = 16

NUM_CORES = 2
NUM_SUBCORES = 16
NW = NUM_CORES * NUM_SUBCORES
BATCH_PER_W = BATCH // NW
CB = 2
CROWS = CB * SEQ
NCHUNK = BATCH_PER_W // CB
ROWS_PER_W = BATCH_PER_W * SEQ
NB = 3


def _sc_body(x_hbm, pidx_hbm, tab_hbm, pos_hbm, out_hbm, idx_v, pidx_v,
             pshared, buf_v, sem_g, sem_p, sem_o):
    wid = lax.axis_index("s") * NUM_CORES + lax.axis_index("c")
    row0 = wid * ROWS_PER_W

    def fill_body(k, c):
        pltpu.sync_copy(pidx_hbm, idx_v.at[pl.ds(k * CROWS, CROWS)])
        return c
    lax.fori_loop(0, ROWS_PER_W // CROWS, fill_body, 0)
    pltpu.sync_copy(pidx_hbm, pidx_v)
    @pl.when(lax.axis_index("s") == 0)
    def _():
        pltpu.sync_copy(pos_hbm, pshared)
    plsc.subcore_barrier()

    def g_start(j, slot):
        pltpu.async_copy(tab_hbm.at[idx_v.at[pl.ds(CROWS * j, CROWS)]],
                         buf_v.at[slot], sem_g.at[slot])

    def g_wait(slot):
        pltpu.make_async_copy(tab_hbm.at[idx_v.at[pl.ds(0, CROWS)]],
                              buf_v.at[slot], sem_g.at[slot]).wait()

    def o_start(j, slot):
        pltpu.async_copy(buf_v.at[slot],
                         out_hbm.at[pl.ds(row0 + CROWS * j, CROWS)],
                         sem_o.at[slot])

    def o_wait(j, slot):
        pltpu.make_async_copy(buf_v.at[slot],
                              out_hbm.at[pl.ds(row0 + CROWS * j, CROWS)],
                              sem_o.at[slot]).wait()

    def add_pos(slot):
        pltpu.async_copy(pshared.at[pidx_v], buf_v.at[slot], sem_p.at[0],
                         add=True)
        pltpu.make_async_copy(pshared.at[pidx_v], buf_v.at[slot],
                              sem_p.at[0]).wait()

    g_start(0, 0)
    g_start(1, 1)
    g_wait(0)
    add_pos(0)
    o_start(0, 0)
    g_start(2, 2)

    def body(j, carry):
        slot0 = lax.rem(j, NB)
        slot2 = lax.rem(j + 2, NB)
        g_wait(slot0)
        add_pos(slot0)
        o_start(j, slot0)
        o_wait(j - 1, slot2)
        g_start(j + 2, slot2)
        return carry

    lax.fori_loop(1, NCHUNK - 2, body, 0)

    j = NCHUNK - 2
    g_wait(j % NB)
    add_pos(j % NB)
    o_start(j, j % NB)
    o_wait(j - 1, (j + 2) % NB)
    j = NCHUNK - 1
    g_wait(j % NB)
    add_pos(j % NB)
    o_start(j, j % NB)
    o_wait(NCHUNK - 2, (NCHUNK - 2) % NB)
    o_wait(NCHUNK - 1, (NCHUNK - 1) % NB)


@jax.jit
def kernel(x, token_table, pos_table):
    x_flat = x.reshape(BATCH * SEQ)
    pos_idx = jnp.tile(jnp.arange(SEQ, dtype=jnp.int32), CB)
    mesh = plsc.VectorSubcoreMesh(core_axis_name="c", subcore_axis_name="s")
    f = pl.kernel(
        _sc_body,
        out_type=jax.ShapeDtypeStruct((BATCH * SEQ, EMBED), jnp.float32),
        mesh=mesh,
        compiler_params=pltpu.CompilerParams(use_tc_tiling_on_sc=False),
        scratch_types=[
            pltpu.VMEM((ROWS_PER_W,), jnp.int32),
            pltpu.VMEM((CROWS,), jnp.int32),
            pltpu.VMEM_SHARED((SEQ, EMBED), jnp.float32),
            pltpu.VMEM((NB, CROWS, EMBED), jnp.float32),
            pltpu.SemaphoreType.DMA((NB,)),
            pltpu.SemaphoreType.DMA((1,)),
            pltpu.SemaphoreType.DMA((NB,)),
        ],
    )
    out = f(x_flat, pos_idx, token_table, pos_table)
    return out.reshape(BATCH, SEQ, EMBED)


# speedup vs baseline: 1.5284x; 1.5284x over previous
import numpy as np

import jax
import jax.numpy as jnp
from jax import lax
from jax.experimental import pallas as pl
from jax.experimental.pallas import tpu as pltpu
from jax.experimental.pallas import tpu_sc as plsc

BATCH = 4096
SEQ = 200
EMBED = 64
LANES = 16

NUM_CORES = 2
NUM_SUBCORES = 16
NW = NUM_CORES * NUM_SUBCORES
BATCH_PER_W = BATCH // NW
NCHUNK = BATCH_PER_W
ROWS_PER_W = BATCH_PER_W * SEQ
NBG = 3
NBO = 2


def _sc_body(x_hbm, tab_hbm, pos_hbm, out_hbm, idx_v, pos_v, gbuf, obuf,
             sem_g, sem_o):
    wid = lax.axis_index("s") * NUM_CORES + lax.axis_index("c")
    row0 = wid * ROWS_PER_W

    pltpu.sync_copy(x_hbm.at[pl.ds(row0, ROWS_PER_W)], idx_v)
    pltpu.sync_copy(pos_hbm, pos_v)

    def g_start(j, slot):
        pltpu.async_copy(tab_hbm.at[idx_v.at[pl.ds(SEQ * j, SEQ)]],
                         gbuf.at[slot], sem_g.at[slot])

    def g_wait(slot):
        pltpu.make_async_copy(tab_hbm.at[idx_v.at[pl.ds(0, SEQ)]],
                              gbuf.at[slot], sem_g.at[slot]).wait()

    def o_start(j, slot):
        pltpu.async_copy(obuf.at[slot],
                         out_hbm.at[pl.ds(row0 + SEQ * j, SEQ)],
                         sem_o.at[slot])

    def o_wait(j, slot):
        pltpu.make_async_copy(obuf.at[slot],
                              out_hbm.at[pl.ds(row0 + SEQ * j, SEQ)],
                              sem_o.at[slot]).wait()

    def convert_add(sg, so):
        @plsc.parallel_loop(0, SEQ, unroll=2)
        def r_body(r):
            for h in range(2):
                w = plsc.bitcast(gbuf[sg, r, pl.ds(32 * h, 32)], jnp.int32)
                lo = plsc.bitcast(lax.shift_left(w, 16), jnp.float32)
                hi = plsc.bitcast(
                    w & jnp.int32(np.int32(np.uint32(0xFFFF0000))),
                    jnp.float32)
                sl0 = pl.ds(32 * h, LANES)
                sl1 = pl.ds(32 * h + LANES, LANES)
                obuf[so, r, sl0] = lo + pos_v[r, sl0]
                obuf[so, r, sl1] = hi + pos_v[r, sl1]

    g_start(0, 0)
    g_start(1, 1)
    g_wait(0)
    convert_add(0, 0)
    o_start(0, 0)
    g_start(2, 2)
    g_wait(1)
    convert_add(1, 1)
    o_start(1, 1)
    g_start(3, 0)

    def body(j, carry):
        sg = lax.rem(j, NBG)
        so = lax.rem(j, NBO)
        g_wait(sg)
        o_wait(j - 2, so)
        convert_add(sg, so)
        o_start(j, so)
        g_start(j + 2, lax.rem(j + 2, NBG))
        return carry

    lax.fori_loop(2, NCHUNK - 2, body, 0)

    j = NCHUNK - 2
    g_wait(j % NBG)
    o_wait(j - 2, j % NBO)
    convert_add(j % NBG, j % NBO)
    o_start(j, j % NBO)
    j = NCHUNK - 1
    g_wait(j % NBG)
    o_wait(j - 2, j % NBO)
    convert_add(j % NBG, j % NBO)
    o_start(j, j % NBO)
    o_wait(NCHUNK - 2, (NCHUNK - 2) % NBO)
    o_wait(NCHUNK - 1, (NCHUNK - 1) % NBO)


@jax.jit
def kernel(x, token_table, pos_table):
    x_flat = x.reshape(BATCH * SEQ)
    tab_bf = (token_table.astype(jnp.bfloat16)
              .reshape(-1, 2, 2, LANES)
              .transpose(0, 1, 3, 2)
              .reshape(-1, EMBED))
    mesh = plsc.VectorSubcoreMesh(core_axis_name="c", subcore_axis_name="s")
    f = pl.kernel(
        _sc_body,
        out_type=jax.ShapeDtypeStruct((BATCH * SEQ, EMBED), jnp.float32),
        mesh=mesh,
        compiler_params=pltpu.CompilerParams(use_tc_tiling_on_sc=False,
                                             needs_layout_passes=False),
        scratch_types=[
            pltpu.VMEM((ROWS_PER_W,), jnp.int32),
            pltpu.VMEM((SEQ, EMBED), jnp.float32),
            pltpu.VMEM((NBG, SEQ, EMBED), jnp.bfloat16),
            pltpu.VMEM((NBO, SEQ, EMBED), jnp.float32),
            pltpu.SemaphoreType.DMA((NBG,)),
            pltpu.SemaphoreType.DMA((NBO,)),
        ],
    )
    out = f(x_flat, tab_bf, pos_table)
    return out.reshape(BATCH, SEQ, EMBED)


# speedup vs baseline: 1.6789x; 1.0984x over previous
import jax
import jax.numpy as jnp
from jax import lax
from jax.experimental import pallas as pl
from jax.experimental.pallas import tpu as pltpu
from jax.experimental.pallas import tpu_sc as plsc

BATCH = 4096
SEQ = 200
EMBED = 64
LANES = 16

NUM_CORES = 2
NUM_SUBCORES = 16
NW = NUM_CORES * NUM_SUBCORES
BATCH_PER_W = BATCH // NW
CB = 2
CROWS = CB * SEQ
NCHUNK = BATCH_PER_W // CB
ROWS_PER_W = BATCH_PER_W * SEQ
NB = 3


def _sc_body(x_hbm, tab_hbm, pos_hbm, out_hbm, idx_v, pos_v, buf_v,
             sem_g, sem_o):
    wid = lax.axis_index("s") * NUM_CORES + lax.axis_index("c")
    row0 = wid * ROWS_PER_W

    pltpu.sync_copy(x_hbm.at[pl.ds(row0, ROWS_PER_W)], idx_v)
    pltpu.sync_copy(pos_hbm, pos_v)

    def g_start(j, slot):
        pltpu.async_copy(tab_hbm.at[idx_v.at[pl.ds(CROWS * j, CROWS)]],
                         buf_v.at[slot], sem_g.at[slot])

    def g_wait(slot):
        pltpu.make_async_copy(tab_hbm.at[idx_v.at[pl.ds(0, CROWS)]],
                              buf_v.at[slot], sem_g.at[slot]).wait()

    def o_start(j, slot):
        pltpu.async_copy(buf_v.at[slot],
                         out_hbm.at[pl.ds(row0 + CROWS * j, CROWS)],
                         sem_o.at[slot])

    def o_wait(j, slot):
        pltpu.make_async_copy(buf_v.at[slot],
                              out_hbm.at[pl.ds(row0 + CROWS * j, CROWS)],
                              sem_o.at[slot]).wait()

    def add_pos(slot):
        @plsc.parallel_loop(0, SEQ, unroll=4)
        def r_body(r):
            for jj in range(EMBED // LANES):
                sl = pl.ds(jj * LANES, LANES)
                p = pos_v[r, sl]
                for b in range(CB):
                    buf_v[slot, b * SEQ + r, sl] = (
                        buf_v[slot, b * SEQ + r, sl] + p)

    g_start(0, 0)
    g_start(1, 1)
    g_wait(0)
    add_pos(0)
    o_start(0, 0)
    g_start(2, 2)

    def body(j, carry):
        slot0 = lax.rem(j, NB)
        slot2 = lax.rem(j + 2, NB)
        g_wait(slot0)
        add_pos(slot0)
        o_start(j, slot0)
        o_wait(j - 1, slot2)
        g_start(j + 2, slot2)
        return carry

    lax.fori_loop(1, NCHUNK - 2, body, 0)

    j = NCHUNK - 2
    g_wait(j % NB)
    add_pos(j % NB)
    o_start(j, j % NB)
    o_wait(j - 1, (j + 2) % NB)
    j = NCHUNK - 1
    g_wait(j % NB)
    add_pos(j % NB)
    o_start(j, j % NB)
    o_wait(NCHUNK - 2, (NCHUNK - 2) % NB)
    o_wait(NCHUNK - 1, (NCHUNK - 1) % NB)


@jax.jit
def kernel(x, token_table, pos_table):
    x_flat = x.reshape(BATCH * SEQ)
    mesh = plsc.VectorSubcoreMesh(core_axis_name="c", subcore_axis_name="s")
    f = pl.kernel(
        _sc_body,
        out_type=jax.ShapeDtypeStruct((BATCH * SEQ, EMBED), jnp.float32),
        mesh=mesh,
        compiler_params=pltpu.CompilerParams(use_tc_tiling_on_sc=False),
        scratch_types=[
            pltpu.VMEM((ROWS_PER_W,), jnp.int32),
            pltpu.VMEM((SEQ, EMBED), jnp.float32),
            pltpu.VMEM((NB, CROWS, EMBED), jnp.float32),
            pltpu.SemaphoreType.DMA((NB,)),
            pltpu.SemaphoreType.DMA((NB,)),
        ],
    )
    out = f(x_flat, token_table, pos_table)
    return out.reshape(BATCH, SEQ, EMBED)
